# baseline (speedup 1.0000x reference)
import functools

import jax
import jax.numpy as jnp
from jax.experimental import pallas as pl
import jax.experimental.pallas.tpu as pltpu

COMMIT_W = 0.1


def _rvq_kernel(x_ref, cb_ref, xq_ref, codes_ref, loss_ref, subq_ref,
                dist_ref, res_ref):
    i = pl.program_id(0)
    q = pl.program_id(1)
    K = cb_ref.shape[1]

    @pl.when(q == 0)
    def _():
        res_ref[...] = x_ref[0]

    residual = res_ref[...]
    cb = cb_ref[q]

    r2 = jnp.sum(residual * residual, axis=1, keepdims=True)
    cb2 = jnp.sum(cb * cb, axis=1)
    xe = jax.lax.dot_general(residual, cb, (((1,), (1,)), ((), ())),
                             preferred_element_type=jnp.float32)
    d = (r2 - 2.0 * xe) + cb2[None, :]

    @pl.when(q == 0)
    def _():
        dist_ref[0] = d

    min_d = jnp.min(d, axis=1, keepdims=True)
    lane = jax.lax.broadcasted_iota(jnp.int32, d.shape, 1)
    idx = jnp.min(jnp.where(d == min_d, lane, K), axis=1)
    codes_ref[...] = idx.reshape(codes_ref.shape)

    onehot = (lane == idx[:, None]).astype(jnp.bfloat16)
    cb_hi = cb.astype(jnp.bfloat16)
    r1 = cb - cb_hi.astype(jnp.float32)
    cb_mid = r1.astype(jnp.bfloat16)
    cb_lo = (r1 - cb_mid.astype(jnp.float32)).astype(jnp.bfloat16)
    dn = (((1,), (0,)), ((), ()))
    q_hi = jax.lax.dot_general(onehot, cb_hi, dn,
                               preferred_element_type=jnp.float32)
    q_mid = jax.lax.dot_general(onehot, cb_mid, dn,
                                preferred_element_type=jnp.float32)
    q_lo = jax.lax.dot_general(onehot, cb_lo, dn,
                               preferred_element_type=jnp.float32)
    quant = (q_hi + q_mid) + q_lo
    subq_ref[0, 0] = quant

    quant_st = residual + (quant - residual)

    @pl.when(q == 0)
    def _():
        xq_ref[0] = quant_st

    @pl.when(q > 0)
    def _():
        xq_ref[0] += quant_st

    partial = jnp.sum(min_d).reshape(1, 1)

    @pl.when((i == 0) & (q == 0))
    def _():
        loss_ref[...] = partial

    @pl.when((i > 0) | (q > 0))
    def _():
        loss_ref[...] = loss_ref[...] + partial

    res_ref[...] = residual - quant


@functools.partial(jax.jit, static_argnames=("block_n",))
def _rvq_forward(x, codebooks, block_n=None):
    B, T, D = x.shape
    NQ, K, _ = codebooks.shape
    N = B * T
    BN = T if block_n is None else block_n
    nsub = T // BN
    nblk = B * nsub

    grid = (nblk, NQ)
    xq, codes, loss, subq, dist = pl.pallas_call(
        _rvq_kernel,
        grid=grid,
        in_specs=[
            pl.BlockSpec((1, BN, D), lambda i, q: (i // nsub, i % nsub, 0)),
            pl.BlockSpec((NQ, K, D), lambda i, q: (0, 0, 0)),
        ],
        out_specs=[
            pl.BlockSpec((1, BN, D), lambda i, q: (i // nsub, i % nsub, 0)),
            pl.BlockSpec((1, 1, 1, BN), lambda i, q: (q, i, 0, 0)),
            pl.BlockSpec((1, 1), lambda i, q: (0, 0)),
            pl.BlockSpec((1, 1, BN, D),
                         lambda i, q: (q, i // nsub, i % nsub, 0)),
            pl.BlockSpec((1, BN, K), lambda i, q: (i // nsub, i % nsub, 0)),
        ],
        out_shape=[
            jax.ShapeDtypeStruct((B, T, D), jnp.float32),
            jax.ShapeDtypeStruct((NQ, nblk, 1, BN), jnp.int32),
            jax.ShapeDtypeStruct((1, 1), jnp.float32),
            jax.ShapeDtypeStruct((NQ, B, T, D), jnp.float32),
            jax.ShapeDtypeStruct((B, T, K), jnp.float32),
        ],
        scratch_shapes=[pltpu.VMEM((BN, D), jnp.float32)],
        compiler_params=pltpu.CompilerParams(
            vmem_limit_bytes=100 * 1024 * 1024),
    )(x, codebooks)

    commit_loss = (loss[0, 0] * (COMMIT_W / (N * D))).astype(jnp.float32)
    return (xq,
            codes.reshape(NQ, B, T),
            commit_loss,
            subq,
            dist)


def kernel(x, codebooks):
    return _rvq_forward(x, codebooks)


# speedup vs baseline: 1.4220x; 1.4220x over previous
import functools

import jax
import jax.numpy as jnp
from jax.experimental import pallas as pl
import jax.experimental.pallas.tpu as pltpu

COMMIT_W = 0.1


def _rvq_kernel(x_ref, cb_ref, xq_ref, codes_ref, loss_ref, subq_ref,
                dist_ref, res_ref):
    i = pl.program_id(0)
    q = pl.program_id(1)
    K = cb_ref.shape[1]
    BT, B, D = x_ref.shape
    R = BT * B

    @pl.when(q == 0)
    def _():
        res_ref[...] = x_ref[...].reshape(R, D)

    residual = res_ref[...]
    cb = cb_ref[q]

    r2 = jnp.sum(residual * residual, axis=1, keepdims=True)
    cb2 = jnp.sum(cb * cb, axis=1)
    xe = jax.lax.dot_general(residual, cb, (((1,), (1,)), ((), ())),
                             preferred_element_type=jnp.float32)
    d = (r2 - 2.0 * xe) + cb2[None, :]

    @pl.when(q == 0)
    def _():
        dist_ref[...] = d.reshape(BT, B, K)

    min_d = jnp.min(d, axis=1, keepdims=True)
    lane = jax.lax.broadcasted_iota(jnp.int32, d.shape, 1)
    idx = jnp.min(jnp.where(d == min_d, lane, K), axis=1)
    codes_ref[...] = idx.reshape(codes_ref.shape)

    onehot = (lane == idx[:, None]).astype(jnp.bfloat16)
    cb_hi = cb.astype(jnp.bfloat16)
    r1 = cb - cb_hi.astype(jnp.float32)
    cb_mid = r1.astype(jnp.bfloat16)
    cb_lo = (r1 - cb_mid.astype(jnp.float32)).astype(jnp.bfloat16)
    dn = (((1,), (0,)), ((), ()))
    q_hi = jax.lax.dot_general(onehot, cb_hi, dn,
                               preferred_element_type=jnp.float32)
    q_mid = jax.lax.dot_general(onehot, cb_mid, dn,
                                preferred_element_type=jnp.float32)
    q_lo = jax.lax.dot_general(onehot, cb_lo, dn,
                               preferred_element_type=jnp.float32)
    quant = (q_hi + q_mid) + q_lo
    subq_ref[0] = quant.reshape(BT, B, D)

    quant_st = residual + (quant - residual)

    @pl.when(q == 0)
    def _():
        xq_ref[...] = quant_st.reshape(BT, B, D)

    @pl.when(q > 0)
    def _():
        xq_ref[...] += quant_st.reshape(BT, B, D)

    partial = jnp.sum(min_d).reshape(1, 1)

    @pl.when((i == 0) & (q == 0))
    def _():
        loss_ref[...] = partial

    @pl.when((i > 0) | (q > 0))
    def _():
        loss_ref[...] = loss_ref[...] + partial

    res_ref[...] = residual - quant


@functools.partial(jax.jit, static_argnames=("block_t",))
def _rvq_forward(x, codebooks, block_t=None):
    B, T, D = x.shape
    NQ, K, _ = codebooks.shape
    N = B * T
    if block_t is None:
        block_t = max(bt for bt in (1, 2, 5, 10, 25, 50, 75, 100, 150)
                      if T % bt == 0)
    BT = block_t
    nsl = T // BT
    R = BT * B

    x_t = jnp.transpose(x, (1, 0, 2))

    grid = (nsl, NQ)
    xq_t, codes, loss, subq_t, dist_t = pl.pallas_call(
        _rvq_kernel,
        grid=grid,
        in_specs=[
            pl.BlockSpec((BT, B, D), lambda i, q: (i, 0, 0)),
            pl.BlockSpec((NQ, K, D), lambda i, q: (0, 0, 0)),
        ],
        out_specs=[
            pl.BlockSpec((BT, B, D), lambda i, q: (i, 0, 0)),
            pl.BlockSpec((1, 1, 1, R), lambda i, q: (q, i, 0, 0)),
            pl.BlockSpec((1, 1), lambda i, q: (0, 0)),
            pl.BlockSpec((1, BT, B, D), lambda i, q: (q, i, 0, 0)),
            pl.BlockSpec((BT, B, K), lambda i, q: (i, 0, 0)),
        ],
        out_shape=[
            jax.ShapeDtypeStruct((T, B, D), jnp.float32),
            jax.ShapeDtypeStruct((NQ, nsl, 1, R), jnp.int32),
            jax.ShapeDtypeStruct((1, 1), jnp.float32),
            jax.ShapeDtypeStruct((NQ, T, B, D), jnp.float32),
            jax.ShapeDtypeStruct((T, B, K), jnp.float32),
        ],
        scratch_shapes=[pltpu.VMEM((R, D), jnp.float32)],
        compiler_params=pltpu.CompilerParams(
            vmem_limit_bytes=100 * 1024 * 1024),
    )(x_t, codebooks)

    xq = jnp.transpose(xq_t, (1, 0, 2))
    subq = jnp.transpose(subq_t, (0, 2, 1, 3))
    dist = jnp.transpose(dist_t, (1, 0, 2))
    codes = jnp.transpose(codes.reshape(NQ, T, B), (0, 2, 1))
    commit_loss = (loss[0, 0] * (COMMIT_W / (N * D))).astype(jnp.float32)
    return (xq, codes, commit_loss, subq, dist)


def kernel(x, codebooks):
    return _rvq_forward(x, codebooks)
